# Initial kernel scaffold; baseline (speedup 1.0000x reference)
import jax
import jax.numpy as jnp
from jax.experimental import pallas as pl


def kernel(x):
    raise NotImplementedError("write your pallas kernel here")


# speedup vs baseline: 13.6147x; 13.6147x over previous
import functools

import jax
import jax.numpy as jnp
from jax import lax
from jax.experimental import pallas as pl
from jax.experimental.pallas import tpu as pltpu
from jax.experimental.pallas import tpu_sc as plsc

_GROUP = 16
_NUM_WORKERS = 32
_CHUNK = 65536


def _group_sort_flat(n_words):
    words_per_worker = n_words // _NUM_WORKERS
    assert n_words % (_NUM_WORKERS * _CHUNK) == 0
    n_chunks = words_per_worker // _CHUNK

    mesh = plsc.VectorSubcoreMesh(core_axis_name="c", subcore_axis_name="s")

    @functools.partial(
        pl.kernel,
        out_type=jax.ShapeDtypeStruct((n_words,), jnp.float32),
        mesh=mesh,
        scratch_types=[
            pltpu.VMEM((_CHUNK,), jnp.float32),
        ],
        compiler_params=pltpu.CompilerParams(needs_layout_passes=False),
    )
    def sc_sort(x_hbm, out_hbm, buf):
        wid = lax.axis_index("s") * 2 + lax.axis_index("c")
        base = wid * words_per_worker

        def chunk_body(g, _):
            off = base + g * _CHUNK
            pltpu.sync_copy(x_hbm.at[pl.ds(off, _CHUNK)], buf)

            @plsc.parallel_loop(0, _CHUNK, step=_GROUP, unroll=8)
            def _(i):
                v = buf[pl.ds(i, _GROUP)]
                sorted_keys, _ = plsc.sort_key_val(v, v)
                buf[pl.ds(i, _GROUP)] = sorted_keys

            pltpu.sync_copy(buf, out_hbm.at[pl.ds(off, _CHUNK)])
            return 0

        lax.fori_loop(0, n_chunks, chunk_body, 0)

    return sc_sort


def kernel(x):
    shape = x.shape
    n_words = x.size
    flat = x.reshape(n_words)
    out = _group_sort_flat(n_words)(flat)
    return out.reshape(shape)


# speedup vs baseline: 15.4466x; 1.1345x over previous
import functools

import jax
import jax.numpy as jnp
from jax import lax
from jax.experimental import pallas as pl
from jax.experimental.pallas import tpu as pltpu
from jax.experimental.pallas import tpu_sc as plsc

_GROUP = 16
_NUM_WORKERS = 32
_CHUNK = 16384


def _group_sort_flat(n_words):
    words_per_worker = n_words // _NUM_WORKERS
    assert n_words % (_NUM_WORKERS * _CHUNK * 2) == 0
    n_pairs = words_per_worker // (2 * _CHUNK)

    mesh = plsc.VectorSubcoreMesh(core_axis_name="c", subcore_axis_name="s")

    @functools.partial(
        pl.kernel,
        out_type=jax.ShapeDtypeStruct((n_words,), jnp.float32),
        mesh=mesh,
        scratch_types=[
            pltpu.VMEM((_CHUNK,), jnp.float32),
            pltpu.VMEM((_CHUNK,), jnp.float32),
            pltpu.VMEM((_CHUNK,), jnp.float32),
            pltpu.VMEM((_CHUNK,), jnp.float32),
            pltpu.SemaphoreType.DMA,
            pltpu.SemaphoreType.DMA,
            pltpu.SemaphoreType.DMA,
            pltpu.SemaphoreType.DMA,
        ],
        compiler_params=pltpu.CompilerParams(needs_layout_passes=False),
    )
    def sc_sort(x_hbm, out_hbm, in0, in1, out0, out1, sin0, sin1, sout0, sout1):
        wid = lax.axis_index("s") * 2 + lax.axis_index("c")
        base = wid * words_per_worker
        ins = (in0, in1)
        outs = (out0, out1)
        sins = (sin0, sin1)
        souts = (sout0, sout1)

        pltpu.make_async_copy(x_hbm.at[pl.ds(base, _CHUNK)], in0, sin0).start()
        pltpu.make_async_copy(
            x_hbm.at[pl.ds(base + _CHUNK, _CHUNK)], in1, sin1
        ).start()

        def pair_body(t, _):
            for b in range(2):
                g = 2 * t + b
                off = base + g * _CHUNK
                ib, ob, sib, sob = ins[b], outs[b], sins[b], souts[b]

                pltpu.make_async_copy(
                    x_hbm.at[pl.ds(off, _CHUNK)], ib, sib
                ).wait()

                @pl.when(t > 0)
                def _():
                    pltpu.make_async_copy(
                        ob, out_hbm.at[pl.ds(off, _CHUNK)], sob
                    ).wait()

                @plsc.parallel_loop(0, _CHUNK, step=_GROUP, unroll=8)
                def _(i):
                    v = ib[pl.ds(i, _GROUP)]
                    sorted_keys, _ = plsc.sort_key_val(v, v)
                    ob[pl.ds(i, _GROUP)] = sorted_keys

                pltpu.make_async_copy(
                    ob, out_hbm.at[pl.ds(off, _CHUNK)], sob
                ).start()

                @pl.when(t < n_pairs - 1)
                def _():
                    pltpu.make_async_copy(
                        x_hbm.at[pl.ds(off + 2 * _CHUNK, _CHUNK)], ib, sib
                    ).start()

            return 0

        lax.fori_loop(0, n_pairs, pair_body, 0)

        tail = base + (2 * n_pairs - 2) * _CHUNK
        pltpu.make_async_copy(out0, out_hbm.at[pl.ds(tail, _CHUNK)], sout0).wait()
        pltpu.make_async_copy(
            out1, out_hbm.at[pl.ds(tail + _CHUNK, _CHUNK)], sout1
        ).wait()

    return sc_sort


def kernel(x):
    shape = x.shape
    n_words = x.size
    flat = x.reshape(n_words)
    out = _group_sort_flat(n_words)(flat)
    return out.reshape(shape)


# speedup vs baseline: 49.0986x; 3.1786x over previous
import functools

import jax
import jax.numpy as jnp
from jax import lax
from jax.experimental import pallas as pl
from jax.experimental.pallas import tpu as pltpu
from jax.experimental.pallas import tpu_sc as plsc

_GROUP = 16
_NUM_WORKERS = 32
_CHUNK_ROWS = 4


def _group_sort_2d(n_rows, n_cols):
    rows_per_worker = n_rows // _NUM_WORKERS
    assert n_rows % (_NUM_WORKERS * _CHUNK_ROWS * 2) == 0
    n_pairs = rows_per_worker // (2 * _CHUNK_ROWS)

    mesh = plsc.VectorSubcoreMesh(core_axis_name="c", subcore_axis_name="s")

    @functools.partial(
        pl.kernel,
        out_type=jax.ShapeDtypeStruct((n_rows, n_cols), jnp.float32),
        mesh=mesh,
        scratch_types=[
            pltpu.VMEM((_CHUNK_ROWS, n_cols), jnp.float32),
            pltpu.VMEM((_CHUNK_ROWS, n_cols), jnp.float32),
            pltpu.VMEM((_CHUNK_ROWS, n_cols), jnp.float32),
            pltpu.VMEM((_CHUNK_ROWS, n_cols), jnp.float32),
            pltpu.SemaphoreType.DMA,
            pltpu.SemaphoreType.DMA,
            pltpu.SemaphoreType.DMA,
            pltpu.SemaphoreType.DMA,
        ],
        compiler_params=pltpu.CompilerParams(needs_layout_passes=False),
    )
    def sc_sort(x_hbm, out_hbm, in0, in1, out0, out1, sin0, sin1, sout0, sout1):
        wid = lax.axis_index("s") * 2 + lax.axis_index("c")
        base = wid * rows_per_worker
        ins = (in0, in1)
        outs = (out0, out1)
        sins = (sin0, sin1)
        souts = (sout0, sout1)

        pltpu.make_async_copy(
            x_hbm.at[pl.ds(base, _CHUNK_ROWS), :], in0, sin0
        ).start()
        pltpu.make_async_copy(
            x_hbm.at[pl.ds(base + _CHUNK_ROWS, _CHUNK_ROWS), :], in1, sin1
        ).start()

        def pair_body(t, _):
            for b in range(2):
                row = base + (2 * t + b) * _CHUNK_ROWS
                ib, ob, sib, sob = ins[b], outs[b], sins[b], souts[b]

                pltpu.make_async_copy(
                    x_hbm.at[pl.ds(row, _CHUNK_ROWS), :], ib, sib
                ).wait()

                @pl.when(t > 0)
                def _():
                    pltpu.make_async_copy(
                        ob, out_hbm.at[pl.ds(row, _CHUNK_ROWS), :], sob
                    ).wait()

                @plsc.parallel_loop(0, n_cols, step=_GROUP, unroll=2)
                def _(i):
                    for r in range(_CHUNK_ROWS):
                        v = ib[r, pl.ds(i, _GROUP)]
                        sorted_keys, _ = plsc.sort_key_val(v, v)
                        ob[r, pl.ds(i, _GROUP)] = sorted_keys

                pltpu.make_async_copy(
                    ob, out_hbm.at[pl.ds(row, _CHUNK_ROWS), :], sob
                ).start()

                @pl.when(t < n_pairs - 1)
                def _():
                    pltpu.make_async_copy(
                        x_hbm.at[pl.ds(row + 2 * _CHUNK_ROWS, _CHUNK_ROWS), :],
                        ib,
                        sib,
                    ).start()

            return 0

        lax.fori_loop(0, n_pairs, pair_body, 0)

        tail = base + (2 * n_pairs - 2) * _CHUNK_ROWS
        pltpu.make_async_copy(
            out0, out_hbm.at[pl.ds(tail, _CHUNK_ROWS), :], sout0
        ).wait()
        pltpu.make_async_copy(
            out1, out_hbm.at[pl.ds(tail + _CHUNK_ROWS, _CHUNK_ROWS), :], sout1
        ).wait()

    return sc_sort


def kernel(x):
    shape = x.shape
    n_cols = shape[-1]
    n_rows = x.size // n_cols
    x2 = x.reshape(n_rows, n_cols)
    out = _group_sort_2d(n_rows, n_cols)(x2)
    return out.reshape(shape)
